# Initial kernel scaffold; baseline (speedup 1.0000x reference)
import jax
import jax.numpy as jnp
from jax.experimental import pallas as pl


def kernel(vertex_xyz, latent, W_in, b_in, Wc, bc, W_out, b_out, edge_index):
    raise NotImplementedError("write your pallas kernel here")


# speedup vs baseline: 5.8914x; 5.8914x over previous
import dataclasses
import functools

import jax
import jax.numpy as jnp
from jax import lax
from jax.experimental import pallas as pl
from jax.experimental.pallas import tpu as pltpu
from jax.experimental.pallas import tpu_sc as plsc

F32 = jnp.float32
NC = 2
NS = 16
LN = 16
CH = 128
H = 128

@functools.cache
def _sc_mesh():
    return plsc.VectorSubcoreMesh(core_axis_name="c", subcore_axis_name="s")


@functools.cache
def _sc_params():
    cp = pltpu.CompilerParams()
    if "needs_layout_passes" in pltpu.CompilerParams.__dataclass_fields__:
        cp = dataclasses.replace(cp, needs_layout_passes=False)
    return cp


def _prop_call(y, src_p, dst_p):
    B, VP, _ = y.shape
    EP = src_p.shape[0]
    eps = EP // NS
    rps = VP // NS

    @functools.partial(
        pl.kernel,
        out_type=jax.ShapeDtypeStruct((B, VP, H), F32),
        mesh=_sc_mesh(),
        scratch_types=[
            pltpu.VMEM_SHARED((VP, H), F32),
            pltpu.VMEM((CH,), jnp.int32),
            pltpu.VMEM((CH,), jnp.int32),
            pltpu.VMEM((CH, H), F32),
            pltpu.SemaphoreType.DMA,
        ],
        compiler_params=_sc_params(),
    )
    def prop_kernel(y_hbm, src_hbm, dst_hbm, zy_hbm, z_sp, idx_s, idx_d,
                    rows_v, sem):
        c = lax.axis_index("c")
        s = lax.axis_index("s")
        r0 = s * rps
        pltpu.sync_copy(y_hbm.at[c].at[pl.ds(r0, rps)], z_sp.at[pl.ds(r0, rps)])
        plsc.subcore_barrier()

        @pl.loop(0, eps, step=CH)
        def _(e):
            base = s * eps + e
            pltpu.sync_copy(src_hbm.at[pl.ds(base, CH)], idx_s)
            pltpu.sync_copy(dst_hbm.at[pl.ds(base, CH)], idx_d)
            pltpu.async_copy(y_hbm.at[c].at[idx_s], rows_v, sem).wait()
            pltpu.sync_copy(rows_v, z_sp.at[idx_d], add=True)

        plsc.subcore_barrier()
        pltpu.sync_copy(z_sp.at[pl.ds(r0, rps)], zy_hbm.at[c].at[pl.ds(r0, rps)])

    return prop_kernel(y, src_p, dst_p)


_DOT = dict(preferred_element_type=F32, precision=lax.Precision.DEFAULT)


def _k0_body(vx_ref, lat_ref, wlat_ref, w3_ref, bin_ref, dd_ref, wc0_ref,
             y_ref, dis_ref):
    latproj = jnp.dot(lat_ref[...], wlat_ref[...], **_DOT)
    bmask = lax.broadcasted_iota(jnp.int32, latproj.shape, 0) == pl.program_id(0)
    row = jnp.sum(jnp.where(bmask, latproj, 0.0), axis=0, keepdims=True)
    x = jnp.dot(vx_ref[...], w3_ref[...], **_DOT)
    x = jax.nn.relu(x + row + bin_ref[0:1, :])
    deg = dd_ref[0][:, 0:1]
    dis = lax.rsqrt(deg)
    y_ref[0] = jnp.dot(x, wc0_ref[...], **_DOT) * dis
    dis_ref[...] = jnp.broadcast_to(dis, dis_ref.shape)


def _k0_call(vx128, latp, Wlat, W3p, binp, dd, Wc0, BLK, B):
    VP = vx128.shape[0]
    LD = latp.shape[1]
    nb = VP // BLK
    return pl.pallas_call(
        _k0_body,
        grid=(B, nb),
        in_specs=[
            pl.BlockSpec((BLK, H), lambda b, j: (j, 0)),
            pl.BlockSpec((8, LD), lambda b, j: (0, 0)),
            pl.BlockSpec((LD, H), lambda b, j: (0, 0)),
            pl.BlockSpec((H, H), lambda b, j: (0, 0)),
            pl.BlockSpec((8, H), lambda b, j: (0, 0)),
            pl.BlockSpec((1, BLK, H), lambda b, j: (0, j, 0)),
            pl.BlockSpec((H, H), lambda b, j: (0, 0)),
        ],
        out_specs=[
            pl.BlockSpec((1, BLK, H), lambda b, j: (b, j, 0)),
            pl.BlockSpec((BLK, H), lambda b, j: (j, 0)),
        ],
        out_shape=[
            jax.ShapeDtypeStruct((B, VP, H), F32),
            jax.ShapeDtypeStruct((VP, H), F32),
        ],
    )(vx128, latp, Wlat, W3p, binp, dd, Wc0)


def _mid_body(zy_ref, dis_ref, bc_ref, wc_ref, y_ref, *, row):
    dis = dis_ref[...]
    x = jax.nn.relu(zy_ref[0] * dis + bc_ref[row:row + 1, :])
    y_ref[0] = jnp.dot(x, wc_ref[...], **_DOT) * dis


def _mid_call(zy, dis, bcp, Wci, row, BLK):
    B, VP, _ = zy.shape
    nb = VP // BLK
    return pl.pallas_call(
        functools.partial(_mid_body, row=row),
        grid=(B, nb),
        in_specs=[
            pl.BlockSpec((1, BLK, H), lambda b, j: (b, j, 0)),
            pl.BlockSpec((BLK, H), lambda b, j: (j, 0)),
            pl.BlockSpec((8, H), lambda b, j: (0, 0)),
            pl.BlockSpec((H, H), lambda b, j: (0, 0)),
        ],
        out_specs=pl.BlockSpec((1, BLK, H), lambda b, j: (b, j, 0)),
        out_shape=jax.ShapeDtypeStruct((B, VP, H), F32),
    )(zy, dis, bcp, Wci)


def _fin_body(zy_ref, dis_ref, bc_ref, wo_ref, bo_ref, o_ref, *, row):
    x = jax.nn.relu(zy_ref[0] * dis_ref[...] + bc_ref[row:row + 1, :])
    o_ref[0] = jnp.dot(x, wo_ref[...], **_DOT) + bo_ref[0:1, :]


def _fin_call(zy, dis, bcp, Wop, bop, row, BLK):
    B, VP, _ = zy.shape
    nb = VP // BLK
    return pl.pallas_call(
        functools.partial(_fin_body, row=row),
        grid=(B, nb),
        in_specs=[
            pl.BlockSpec((1, BLK, H), lambda b, j: (b, j, 0)),
            pl.BlockSpec((BLK, H), lambda b, j: (j, 0)),
            pl.BlockSpec((8, H), lambda b, j: (0, 0)),
            pl.BlockSpec((H, H), lambda b, j: (0, 0)),
            pl.BlockSpec((8, H), lambda b, j: (0, 0)),
        ],
        out_specs=pl.BlockSpec((1, BLK, H), lambda b, j: (b, j, 0)),
        out_shape=jax.ShapeDtypeStruct((B, VP, H), F32),
    )(zy, dis, bcp, Wop, bop)


def kernel(vertex_xyz, latent, W_in, b_in, Wc, bc, W_out, b_out, edge_index):
    V = vertex_xyz.shape[0]
    B = latent.shape[0]
    LD = latent.shape[1]
    E = edge_index.shape[1]
    L = Wc.shape[0]
    O = W_out.shape[1]
    BLK = 2048
    VP = -(-V // BLK) * BLK
    EP = -(-E // (NS * CH)) * (NS * CH)
    junk = VP - 1

    src_p = jnp.concatenate(
        [edge_index[0], jnp.zeros((EP - E,), jnp.int32)])
    dst_p = jnp.concatenate(
        [edge_index[1], jnp.full((EP - E,), junk, jnp.int32)])
    vx128 = jnp.zeros((VP, H), F32).at[:V, :3].set(vertex_xyz)
    W3p = jnp.zeros((H, H), F32).at[:3].set(W_in[:3])
    Wlat = W_in[3:]
    latp = jnp.zeros((8, LD), F32).at[:B].set(latent)
    binp = jnp.zeros((8, H), F32).at[0].set(b_in)
    bcp = jnp.zeros((8, H), F32).at[:L].set(bc)
    Wop = jnp.zeros((H, H), F32).at[:, :O].set(W_out)
    bop = jnp.zeros((8, H), F32).at[0, :O].set(b_out)

    dd = _prop_call(jnp.ones((B, VP, H), F32), src_p, dst_p)
    y, dis = _k0_call(vx128, latp, Wlat, W3p, binp, dd, Wc[0], BLK, B)
    for i in range(1, L):
        zy = _prop_call(y, src_p, dst_p)
        y = _mid_call(zy, dis, bcp, Wc[i], i - 1, BLK)
    zy = _prop_call(y, src_p, dst_p)
    out = _fin_call(zy, dis, bcp, Wop, bop, L - 1, BLK)
    return out[:, :V, :O]
